# Initial kernel scaffold; baseline (speedup 1.0000x reference)
import jax
import jax.numpy as jnp
from jax.experimental import pallas as pl


def kernel(x):
    raise NotImplementedError("write your pallas kernel here")


# speedup vs baseline: 3.7572x; 3.7572x over previous
import functools

import numpy as np
import jax
import jax.numpy as jnp
from jax.experimental import pallas as pl

L = 2048
F = 1025
FPAD = 1152
C = 128
LEVELS_N = 3
K = 5

_HI = jax.lax.Precision.HIGHEST


def _build_bases():
    t = np.arange(L, dtype=np.int64)
    f = np.arange(FPAD, dtype=np.int64)
    ang = 2.0 * np.pi * ((f[:, None] * t[None, :]) % L).astype(np.float64) / L
    cosf = np.cos(ang)
    sinf = np.sin(ang)
    cosf[F:] = 0.0
    sinf[F:] = 0.0
    scale = np.full(FPAD, 2.0 / L)
    scale[0] = 1.0 / L
    scale[L // 2] = 1.0 / L
    scale[F:] = 0.0
    icos_t = (cosf * scale[:, None]).T.copy()
    isin_t = (sinf * scale[:, None]).T.copy()
    isin_t[:, 0] = 0.0
    isin_t[:, L // 2] = 0.0
    return (cosf.astype(np.float32), sinf.astype(np.float32),
            icos_t.astype(np.float32), isin_t.astype(np.float32))


_COSF, _SINF, _ICOST, _ISINT = _build_bases()


def _topk_keep(mag):
    cnt = jnp.zeros((1, C), jnp.int32)
    keep = jnp.zeros((FPAD, C), jnp.bool_)
    cur = mag
    for _ in range(K):
        v = jnp.max(cur, axis=0, keepdims=True)
        eq = cur == v
        c = jnp.sum(eq.astype(jnp.int32), axis=0, keepdims=True)
        newcnt = cnt + c
        keep = jnp.logical_or(keep, jnp.logical_and(eq, newcnt < K))
        cnt = jnp.where(cnt < K, newcnt, cnt)
        cur = jnp.where(eq, -jnp.inf, cur)
    return keep


def _fwd_select_kernel(x_ref, cos_ref, sin_ref,
                       ar1, ai1, ar2, ai2, ar3, ai3):
    x = x_ref[0]
    re = jnp.dot(cos_ref[...], x, precision=_HI,
                 preferred_element_type=jnp.float32)
    im = -jnp.dot(sin_ref[...], x, precision=_HI,
                  preferred_element_type=jnp.float32)
    mag = re * re + im * im
    lane = jax.lax.broadcasted_iota(jnp.int32, (FPAD, C), 1)
    mag = jnp.where(lane == 0, 0.0, mag)
    outs = ((ar1, ai1), (ar2, ai2), (ar3, ai3))
    for lvl in range(LEVELS_N):
        keep = _topk_keep(mag)
        outs[lvl][0][0] = jnp.where(keep, re, 0.0)
        outs[lvl][1][0] = jnp.where(keep, im, 0.0)
        mag = jnp.where(keep, 0.0, mag)


TBLK = 256


def _synth_kernel(x_ref, ic_ref, is_ref,
                  ar1, ai1, ar2, ai2, ar3, ai3,
                  s1, s2, s3, r1, r2, r3):
    tb = pl.program_id(1)
    rows = pl.ds(tb * TBLK, TBLK)
    ic = ic_ref[rows, :]
    isn = is_ref[rows, :]
    r = x_ref[0]
    ins = ((ar1, ai1), (ar2, ai2), (ar3, ai3))
    souts = (s1, s2, s3)
    routs = (r1, r2, r3)
    for lvl in range(LEVELS_N):
        ar = ins[lvl][0][0]
        ai = ins[lvl][1][0]
        s = (jnp.dot(ic, ar, precision=_HI, preferred_element_type=jnp.float32)
             - jnp.dot(isn, ai, precision=_HI,
                       preferred_element_type=jnp.float32))
        souts[lvl][0] = s
        r = r - s
        routs[lvl][0] = r


@functools.partial(jax.jit, static_argnums=())
def kernel(x):
    B = x.shape[0]
    spec_bfc = pl.BlockSpec((1, FPAD, C), lambda b: (b, 0, 0))
    spec_x = pl.BlockSpec((1, L, C), lambda b: (b, 0, 0))
    spec_full_fl = pl.BlockSpec((FPAD, L), lambda b: (0, 0))
    spec_full_lf = pl.BlockSpec((L, FPAD), lambda b: (0, 0))
    sds_bfc = jax.ShapeDtypeStruct((B, FPAD, C), jnp.float32)
    sds_blc = jax.ShapeDtypeStruct((B, L, C), jnp.float32)

    ar1, ai1, ar2, ai2, ar3, ai3 = pl.pallas_call(
        _fwd_select_kernel,
        grid=(B,),
        in_specs=[spec_x, spec_full_fl, spec_full_fl],
        out_specs=[spec_bfc] * 6,
        out_shape=[sds_bfc] * 6,
    )(x, _COSF, _SINF)

    spec_xt = pl.BlockSpec((1, TBLK, C), lambda b, t: (b, t, 0))
    spec_lf2 = pl.BlockSpec((L, FPAD), lambda b, t: (0, 0))
    spec_bfc2 = pl.BlockSpec((1, FPAD, C), lambda b, t: (b, 0, 0))
    s1, s2, s3, r1, r2, r3 = pl.pallas_call(
        _synth_kernel,
        grid=(B, L // TBLK),
        in_specs=[spec_xt, spec_lf2, spec_lf2] + [spec_bfc2] * 6,
        out_specs=[spec_xt] * 6,
        out_shape=[sds_blc] * 6,
    )(x, _ICOST, _ISINT, ar1, ai1, ar2, ai2, ar3, ai3)

    return (s1, s2, s3, r1, r2, r3)


# speedup vs baseline: 5.5420x; 1.4751x over previous
import functools

import numpy as np
import jax
import jax.numpy as jnp
from jax.experimental import pallas as pl

L = 2048
F = 1025
FPAD = 1152
C = 128
LEVELS_N = 3
K = 5

_HI = jax.lax.Precision.HIGHEST
_MED = jax.lax.Precision.HIGH


def _build_bases():
    t = np.arange(L, dtype=np.int64)
    f = np.arange(FPAD, dtype=np.int64)
    ang = 2.0 * np.pi * ((f[:, None] * t[None, :]) % L).astype(np.float64) / L
    cosf = np.cos(ang)
    sinf = np.sin(ang)
    cosf[F:] = 0.0
    sinf[F:] = 0.0
    scale = np.full(FPAD, 2.0 / L)
    scale[0] = 1.0 / L
    scale[L // 2] = 1.0 / L
    scale[F:] = 0.0
    icos_t = (cosf * scale[:, None]).T.copy()
    isin_t = (sinf * scale[:, None]).T.copy()
    isin_t[:, 0] = 0.0
    isin_t[:, L // 2] = 0.0

    def _split(m):
        hi = m.astype(np.float32).astype(jnp.bfloat16)
        lo = (m - np.asarray(hi, np.float64)).astype(np.float32)
        lo = lo.astype(jnp.bfloat16)
        return np.asarray(hi), np.asarray(lo)

    ic_hi, ic_lo = _split(icos_t)
    is_hi, is_lo = _split(isin_t)
    return (cosf.astype(np.float32), sinf.astype(np.float32),
            ic_hi, ic_lo, is_hi, is_lo)


_COSF, _SINF, _ICH, _ICL, _ISH, _ISL = _build_bases()


def _topk_keep(mag):
    cnt = jnp.zeros((1, C), jnp.int32)
    keep = jnp.zeros((FPAD, C), jnp.bool_)
    cur = mag
    for _ in range(K):
        v = jnp.max(cur, axis=0, keepdims=True)
        eq = cur == v
        c = jnp.sum(eq.astype(jnp.int32), axis=0, keepdims=True)
        newcnt = cnt + c
        keep = jnp.logical_or(keep, jnp.logical_and(eq, newcnt < K))
        cnt = jnp.where(cnt < K, newcnt, cnt)
        cur = jnp.where(eq, -jnp.inf, cur)
    return keep


def _fwd_select_kernel(x_ref, cos_ref, sin_ref,
                       ar1, ai1, ar2, ai2, ar3, ai3):
    x = x_ref[0]
    re = jnp.dot(cos_ref[...], x, precision=_HI,
                 preferred_element_type=jnp.float32)
    im = -jnp.dot(sin_ref[...], x, precision=_HI,
                  preferred_element_type=jnp.float32)
    mag = re * re + im * im
    lane = jax.lax.broadcasted_iota(jnp.int32, (FPAD, C), 1)
    mag = jnp.where(lane == 0, 0.0, mag)
    outs = ((ar1, ai1), (ar2, ai2), (ar3, ai3))
    for lvl in range(LEVELS_N):
        keep = _topk_keep(mag)
        outs[lvl][0][0] = jnp.where(keep, re, 0.0)
        outs[lvl][1][0] = jnp.where(keep, im, 0.0)
        mag = jnp.where(keep, 0.0, mag)


TBLK = 256


def _dot3(bhi_ref, blo_ref, rows, a):
    bhi = bhi_ref[rows, :]
    blo = blo_ref[rows, :]
    a_hi = a.astype(jnp.bfloat16)
    a_lo = (a - a_hi.astype(jnp.float32)).astype(jnp.bfloat16)
    acc = jnp.dot(bhi, a_hi, preferred_element_type=jnp.float32)
    acc += jnp.dot(bhi, a_lo, preferred_element_type=jnp.float32)
    acc += jnp.dot(blo, a_hi, preferred_element_type=jnp.float32)
    return acc


def _synth_kernel(x_ref, ich_ref, icl_ref, ish_ref, isl_ref,
                  ar1, ai1, ar2, ai2, ar3, ai3,
                  s1, s2, s3, r1, r2, r3):
    tb = pl.program_id(1)
    rows = pl.ds(tb * TBLK, TBLK)
    r = x_ref[0]
    ins = ((ar1, ai1), (ar2, ai2), (ar3, ai3))
    souts = (s1, s2, s3)
    routs = (r1, r2, r3)
    for lvl in range(LEVELS_N):
        ar = ins[lvl][0][0]
        ai = ins[lvl][1][0]
        s = (_dot3(ich_ref, icl_ref, rows, ar)
             - _dot3(ish_ref, isl_ref, rows, ai))
        souts[lvl][0] = s
        r = r - s
        routs[lvl][0] = r


@functools.partial(jax.jit, static_argnums=())
def kernel(x):
    B = x.shape[0]
    spec_bfc = pl.BlockSpec((1, FPAD, C), lambda b: (b, 0, 0))
    spec_x = pl.BlockSpec((1, L, C), lambda b: (b, 0, 0))
    spec_full_fl = pl.BlockSpec((FPAD, L), lambda b: (0, 0))
    spec_full_lf = pl.BlockSpec((L, FPAD), lambda b: (0, 0))
    sds_bfc = jax.ShapeDtypeStruct((B, FPAD, C), jnp.float32)
    sds_blc = jax.ShapeDtypeStruct((B, L, C), jnp.float32)

    ar1, ai1, ar2, ai2, ar3, ai3 = pl.pallas_call(
        _fwd_select_kernel,
        grid=(B,),
        in_specs=[spec_x, spec_full_fl, spec_full_fl],
        out_specs=[spec_bfc] * 6,
        out_shape=[sds_bfc] * 6,
    )(x, _COSF, _SINF)

    spec_xt = pl.BlockSpec((1, TBLK, C), lambda b, t: (b, t, 0))
    spec_lf2 = pl.BlockSpec((L, FPAD), lambda b, t: (0, 0))
    spec_bfc2 = pl.BlockSpec((1, FPAD, C), lambda b, t: (b, 0, 0))
    s1, s2, s3, r1, r2, r3 = pl.pallas_call(
        _synth_kernel,
        grid=(B, L // TBLK),
        in_specs=[spec_xt] + [spec_lf2] * 4 + [spec_bfc2] * 6,
        out_specs=[spec_xt] * 6,
        out_shape=[sds_blc] * 6,
    )(x, _ICH, _ICL, _ISH, _ISL, ar1, ai1, ar2, ai2, ar3, ai3)

    return (s1, s2, s3, r1, r2, r3)
